# baseline (speedup 1.0000x reference)
import numpy as np
import jax
import jax.numpy as jnp
from jax import lax
from jax.experimental import pallas as pl
from jax.experimental.pallas import tpu as pltpu
from jax.experimental.pallas import tpu_sc as plsc

_B = 16384
_E = 128
_NC, _NS = 2, 16
_NW = _NC * _NS
_BPW = _B // _NW
_CH = 128
_NCH = _BPW // _CH

_PERM = np.empty(_E, np.int32)
for _g in range(_E // 32):
    for _i in range(16):
        _PERM[32 * _g + 2 * _i] = 32 * _g + _i
        _PERM[32 * _g + 2 * _i + 1] = 32 * _g + 16 + _i


_RND = jnp.uint32(0x8000)
_HI = jnp.uint32(0xFFFF0000)


def _convert_chunk(rows_v, rows_pk, j):
    def row_body(r, carry):
        for u in range(2):
            rr = j * _CH + 2 * r + u
            for g in range(_E // 32):
                a = rows_v[rr, pl.ds(32 * g, 16)]
                b = rows_v[rr, pl.ds(32 * g + 16, 16)]
                y = ((a + _RND) >> 16) | ((b + _RND) & _HI)
                rows_pk[rr, pl.ds(16 * g, 16)] = y
        return carry

    lax.fori_loop(0, _CH // 2, row_body, 0)


def _sc_gather_body(u_idx, m_idx, u_tab, m_tab, out_u, out_m,
                    idx_u, idx_m, rows_v, rows_pk, sem_g, sem_w):
    wid = lax.axis_index("s") * _NC + lax.axis_index("c")
    base = wid * _BPW

    pltpu.sync_copy(u_idx.at[pl.ds(wid * _NCH, _NCH)], idx_u)
    pltpu.sync_copy(m_idx.at[pl.ds(wid * _NCH, _NCH)], idx_m)

    chunks = [(u_tab, idx_u, out_u, j) for j in range(_NCH)] + \
             [(m_tab, idx_m, out_m, j) for j in range(_NCH)]

    def fire_gather(k):
        tab, idx, _, j = chunks[k]
        slot = (k % 2) * _CH
        return pltpu.async_copy(tab.at[idx.at[j]],
                                rows_v.at[pl.ds(slot, _CH)], sem_g)

    g = {0: fire_gather(0), 1: fire_gather(1)}
    w = {}
    for k in range(len(chunks)):
        slot = k % 2
        _, _, out, j = chunks[k]
        g[k].wait()
        if k >= 2:
            w[k - 2].wait()
        _convert_chunk(rows_v, rows_pk, slot)
        if k + 2 < len(chunks):
            g[k + 2] = fire_gather(k + 2)
        w[k] = pltpu.async_copy(rows_pk.at[pl.ds(slot * _CH, _CH)],
                                out.at[pl.ds(base + j * _CH, _CH)],
                                sem_w)
    w[len(chunks) - 2].wait()
    w[len(chunks) - 1].wait()


def _make_sc_gather():
    return pl.kernel(
        _sc_gather_body,
        out_type=(jax.ShapeDtypeStruct((_B, _E // 2), jnp.uint32),
                  jax.ShapeDtypeStruct((_B, _E // 2), jnp.uint32)),
        mesh=plsc.VectorSubcoreMesh(core_axis_name="c", subcore_axis_name="s",
                                    num_cores=_NC, num_subcores=_NS),
        scratch_types=[
            pltpu.VMEM((_NCH, _CH), jnp.int32),
            pltpu.VMEM((_NCH, _CH), jnp.int32),
            pltpu.VMEM((2 * _CH, _E), jnp.uint32),
            pltpu.VMEM((2 * _CH, _E // 2), jnp.uint32),
            pltpu.SemaphoreType.DMA,
            pltpu.SemaphoreType.DMA,
        ],
    )


_BLK = 4096


def _mlp_body(ue, me, w1a, w1b, b1, w2, b2, w3, b3, out):
    x = jnp.dot(ue[...], w1a[...], preferred_element_type=jnp.float32)
    x = x + jnp.dot(me[...], w1b[...], preferred_element_type=jnp.float32)
    x = jnp.maximum(x + b1[...], 0.0)
    x = jnp.maximum(
        jnp.dot(x, w2[...], preferred_element_type=jnp.float32) + b2[...], 0.0)
    out[...] = jnp.dot(x, w3[...], preferred_element_type=jnp.float32) + b3[...]


def _mlp_call(ue, me, w1a, w1b, b1, w2, b2, w3, b3):
    grid = (_B // _BLK,)
    wspec = lambda shape: pl.BlockSpec(shape, lambda i: (0, 0))
    return pl.pallas_call(
        _mlp_body,
        grid=grid,
        in_specs=[
            pl.BlockSpec((_BLK, _E), lambda i: (i, 0)),
            pl.BlockSpec((_BLK, _E), lambda i: (i, 0)),
            wspec((_E, 128)),
            wspec((_E, 128)),
            wspec((1, 128)),
            wspec((128, 64)),
            wspec((1, 64)),
            wspec((64, 1)),
            wspec((1, 1)),
        ],
        out_specs=pl.BlockSpec((_BLK, 1), lambda i: (i, 0)),
        out_shape=jax.ShapeDtypeStruct((_B, 1), jnp.float32),
    )(ue, me, w1a, w1b, b1, w2, b2, w3, b3)


def kernel(user, movie, user_table, movie_table, W1, b1, W2, b2, W3, b3):
    u2 = user.reshape(_B // _CH, _CH)
    m2 = movie.reshape(_B // _CH, _CH)
    ut_u32 = lax.bitcast_convert_type(user_table, jnp.uint32)
    mt_u32 = lax.bitcast_convert_type(movie_table, jnp.uint32)
    ue_pk, me_pk = _make_sc_gather()(u2, m2, ut_u32, mt_u32)
    ue = lax.bitcast_convert_type(ue_pk, jnp.bfloat16).reshape(_B, _E)
    me = lax.bitcast_convert_type(me_pk, jnp.bfloat16).reshape(_B, _E)
    perm = jnp.asarray(_PERM)
    w1a = W1[:_E][perm, :].astype(jnp.bfloat16)
    w1b = W1[_E:][perm, :].astype(jnp.bfloat16)
    out = _mlp_call(ue, me, w1a, w1b, b1.reshape(1, -1),
                    W2, b2.reshape(1, -1), W3, b3.reshape(1, 1))
    return out[:, 0]


# speedup vs baseline: 3.3631x; 3.3631x over previous
import jax
import jax.numpy as jnp
from jax import lax
from jax.experimental import pallas as pl
from jax.experimental.pallas import tpu as pltpu
from jax.experimental.pallas import tpu_sc as plsc

_B = 16384
_E = 128
_NC, _NS = 2, 16
_NW = _NC * _NS
_CH = 128
_SEG = 2
_BSEG = _B // _SEG


def _make_sc_gather(bseg):
    bpw = bseg // _NW
    nch = bpw // _CH

    def body(u_idx, m_idx, u_tab, m_tab, out_u, out_m,
             idx_u, idx_m, rows_v, sem_g, sem_w):
        wid = lax.axis_index("s") * _NC + lax.axis_index("c")
        base = wid * bpw

        pltpu.sync_copy(u_idx.at[pl.ds(wid * nch, nch)], idx_u)
        pltpu.sync_copy(m_idx.at[pl.ds(wid * nch, nch)], idx_m)

        ug = [pltpu.async_copy(u_tab.at[idx_u.at[j]],
                               rows_v.at[pl.ds(j * _CH, _CH)], sem_g)
              for j in range(nch)]
        uw = []
        for j in range(nch):
            ug[j].wait()
            uw.append(pltpu.async_copy(rows_v.at[pl.ds(j * _CH, _CH)],
                                       out_u.at[pl.ds(base + j * _CH, _CH)],
                                       sem_w))
        mg = []
        for j in range(nch):
            uw[j].wait()
            mg.append(pltpu.async_copy(m_tab.at[idx_m.at[j]],
                                       rows_v.at[pl.ds(j * _CH, _CH)], sem_g))
        mw = []
        for j in range(nch):
            mg[j].wait()
            mw.append(pltpu.async_copy(rows_v.at[pl.ds(j * _CH, _CH)],
                                       out_m.at[pl.ds(base + j * _CH, _CH)],
                                       sem_w))
        for c in mw:
            c.wait()

    return pl.kernel(
        body,
        out_type=(jax.ShapeDtypeStruct((bseg, _E), jnp.float32),
                  jax.ShapeDtypeStruct((bseg, _E), jnp.float32)),
        mesh=plsc.VectorSubcoreMesh(core_axis_name="c", subcore_axis_name="s",
                                    num_cores=_NC, num_subcores=_NS),
        scratch_types=[
            pltpu.VMEM((nch, _CH), jnp.int32),
            pltpu.VMEM((nch, _CH), jnp.int32),
            pltpu.VMEM((bpw, _E), jnp.float32),
            pltpu.SemaphoreType.DMA,
            pltpu.SemaphoreType.DMA,
        ],
    )


_BLK = 4096


def _mlp_body(ue, me, w1a, w1b, b1, w2, b2, w3, b3, out):
    x = jnp.dot(ue[...], w1a[...], preferred_element_type=jnp.float32)
    x = x + jnp.dot(me[...], w1b[...], preferred_element_type=jnp.float32)
    x = jnp.maximum(x + b1[...], 0.0)
    x = jnp.maximum(
        jnp.dot(x, w2[...], preferred_element_type=jnp.float32) + b2[...], 0.0)
    out[...] = jnp.dot(x, w3[...], preferred_element_type=jnp.float32) + b3[...]


def _mlp_call(ue, me, w1a, w1b, b1, w2, b2, w3, b3):
    bseg = ue.shape[0]
    grid = (bseg // _BLK,)
    wspec = lambda shape: pl.BlockSpec(shape, lambda i: (0, 0))
    return pl.pallas_call(
        _mlp_body,
        grid=grid,
        in_specs=[
            pl.BlockSpec((_BLK, _E), lambda i: (i, 0)),
            pl.BlockSpec((_BLK, _E), lambda i: (i, 0)),
            wspec((_E, 128)),
            wspec((_E, 128)),
            wspec((1, 128)),
            wspec((128, 64)),
            wspec((1, 64)),
            wspec((64, 1)),
            wspec((1, 1)),
        ],
        out_specs=pl.BlockSpec((_BLK, 1), lambda i: (i, 0)),
        out_shape=jax.ShapeDtypeStruct((bseg, 1), jnp.float32),
    )(ue, me, w1a, w1b, b1, w2, b2, w3, b3)


def kernel(user, movie, user_table, movie_table, W1, b1, W2, b2, W3, b3):
    rows = _BSEG // _CH
    u2 = user.reshape(_B // _CH, _CH)
    m2 = movie.reshape(_B // _CH, _CH)
    sc = _make_sc_gather(_BSEG)
    gathered = []
    for s in range(_SEG):
        us = lax.slice_in_dim(u2, s * rows, (s + 1) * rows, axis=0)
        ms = lax.slice_in_dim(m2, s * rows, (s + 1) * rows, axis=0)
        gathered.append(sc(us, ms, user_table, movie_table))
    w1a, w1b = W1[:_E], W1[_E:]
    b1r, b2r, b3r = b1.reshape(1, -1), b2.reshape(1, -1), b3.reshape(1, 1)
    outs = [_mlp_call(ue, me, w1a, w1b, b1r, W2, b2r, W3, b3r)
            for ue, me in gathered]
    return jnp.concatenate(outs, axis=0)[:, 0]


# speedup vs baseline: 3.5413x; 1.0530x over previous
import jax
import jax.numpy as jnp
from jax import lax
from jax.experimental import pallas as pl
from jax.experimental.pallas import tpu as pltpu
from jax.experimental.pallas import tpu_sc as plsc

_B = 16384
_E = 128
_NC, _NS = 2, 16
_NW = _NC * _NS
_BPW = _B // _NW
_CH = 128
_NCH = _BPW // _CH


def _sc_gather_body(u_idx, m_idx, u_tab, m_tab, out_u, out_m,
                    idx_v, rows_v, sem):
    wid = lax.axis_index("s") * _NC + lax.axis_index("c")
    base = wid * _BPW

    def one_table(idx_hbm, tab_hbm, out_hbm):
        pltpu.sync_copy(idx_hbm.at[pl.ds(wid * _NCH, _NCH)], idx_v)
        copies = [
            pltpu.async_copy(tab_hbm.at[idx_v.at[j]],
                             rows_v.at[pl.ds(j * _CH, _CH)], sem)
            for j in range(_NCH)
        ]
        for c in copies:
            c.wait()
        pltpu.sync_copy(rows_v, out_hbm.at[pl.ds(base, _BPW)])

    one_table(u_idx, u_tab, out_u)
    one_table(m_idx, m_tab, out_m)


def _make_sc_gather():
    return pl.kernel(
        _sc_gather_body,
        out_type=(jax.ShapeDtypeStruct((_B, _E), jnp.float32),
                  jax.ShapeDtypeStruct((_B, _E), jnp.float32)),
        mesh=plsc.VectorSubcoreMesh(core_axis_name="c", subcore_axis_name="s",
                                    num_cores=_NC, num_subcores=_NS),
        scratch_types=[
            pltpu.VMEM((_NCH, _CH), jnp.int32),
            pltpu.VMEM((_BPW, _E), jnp.float32),
            pltpu.SemaphoreType.DMA,
        ],
    )


_BLK = 4096


def _mlp_body(ue, me, w1a, w1b, b1, w2, b2, w3, b3, out):
    x = jnp.dot(ue[...], w1a[...], preferred_element_type=jnp.float32)
    x = x + jnp.dot(me[...], w1b[...], preferred_element_type=jnp.float32)
    x = jnp.maximum(x + b1[...], 0.0)
    x = jnp.maximum(
        jnp.dot(x, w2[...], preferred_element_type=jnp.float32) + b2[...], 0.0)
    out[...] = jnp.dot(x, w3[...], preferred_element_type=jnp.float32) + b3[...]


def _mlp_call(ue, me, w1a, w1b, b1, w2, b2, w3, b3):
    grid = (_B // _BLK,)
    wspec = lambda shape: pl.BlockSpec(shape, lambda i: (0, 0))
    return pl.pallas_call(
        _mlp_body,
        grid=grid,
        in_specs=[
            pl.BlockSpec((_BLK, _E), lambda i: (i, 0)),
            pl.BlockSpec((_BLK, _E), lambda i: (i, 0)),
            wspec((_E, 128)),
            wspec((_E, 128)),
            wspec((1, 128)),
            wspec((128, 64)),
            wspec((1, 64)),
            wspec((64, 1)),
            wspec((1, 1)),
        ],
        out_specs=pl.BlockSpec((_BLK, 1), lambda i: (i, 0)),
        out_shape=jax.ShapeDtypeStruct((_B, 1), jnp.float32),
    )(ue, me, w1a, w1b, b1, w2, b2, w3, b3)


def kernel(user, movie, user_table, movie_table, W1, b1, W2, b2, W3, b3):
    u2 = user.reshape(_B // _CH, _CH)
    m2 = movie.reshape(_B // _CH, _CH)
    ue, me = _make_sc_gather()(u2, m2, user_table, movie_table)
    out = _mlp_call(ue, me, W1[:_E], W1[_E:], b1.reshape(1, -1),
                    W2, b2.reshape(1, -1), W3, b3.reshape(1, 1))
    return out[:, 0]
